# Initial kernel scaffold; baseline (speedup 1.0000x reference)
import jax
import jax.numpy as jnp
from jax.experimental import pallas as pl


def kernel(x, fixed_keys, fixed_values, W1, b1, W2, b2):
    raise NotImplementedError("write your pallas kernel here")


# speedup vs baseline: 16.8750x; 16.8750x over previous
import jax
import jax.numpy as jnp
from jax import lax
from jax.experimental import pallas as pl
from jax.experimental.pallas import tpu as pltpu

B = 16384
IN_DIM = 128
EMB_DIM = 128
K_FIXED = 64
HIDDEN = 4

BLK = 1024
GRID = B // BLK


def _canon_bits(v):
    return jnp.where(v == 0.0, 0, lax.bitcast_convert_type(v, jnp.int32))


def _body(x_ref, keys_ref, keys_t_ref, vals_ref, w1_ref, b1_ref, w2_ref,
          b2_ref, out_ref, found_sc, fix_sc):
    x = x_ref[...]
    keys = keys_ref[...]

    h = jnp.maximum(
        jnp.dot(x, w1_ref[...], preferred_element_type=jnp.float32)
        + b1_ref[...], 0.0)
    net = (jnp.dot(h, w2_ref[...], preferred_element_type=jnp.float32)
           + b2_ref[...])

    row_hash = jnp.sum(_canon_bits(x), axis=1, keepdims=True)
    key_hash = jnp.sum(_canon_bits(keys_t_ref[...]), axis=0,
                       keepdims=True)
    cand = row_hash == key_hash

    out_ref[...] = net

    @pl.when(jnp.any(cand))
    def _verify():
        iota = lax.broadcasted_iota(jnp.int32, (BLK, K_FIXED), 1)
        first = jnp.min(jnp.where(cand, iota, K_FIXED), axis=1,
                        keepdims=True)
        onehot = (iota == first) & cand
        has_cand = jnp.any(onehot, axis=1, keepdims=True)
        oh_f = onehot.astype(jnp.float32)
        gk = jnp.dot(oh_f, keys, preferred_element_type=jnp.float32,
                     precision=lax.Precision.HIGHEST)
        rowok = (jnp.all(x == gk, axis=1, keepdims=True)
                 & has_cand)
        fixed1 = jnp.dot(oh_f * rowok.astype(jnp.float32), vals_ref[...],
                         preferred_element_type=jnp.float32,
                         precision=lax.Precision.HIGHEST)
        out_ref[...] = jnp.where(rowok, fixed1, net)

        leftover = cand & jnp.logical_not(onehot)

        @pl.when(jnp.any(leftover & jnp.logical_not(rowok)))
        def _fallback():
            found_sc[...] = jnp.zeros((BLK, 1), jnp.float32)
            fix_sc[...] = jnp.zeros((BLK, EMB_DIM), jnp.float32)
            ones_col = jnp.ones((IN_DIM, 1), jnp.float32)

            def scan_key(j, carry):
                keyj = keys_ref[pl.ds(j, 1), :]
                eq = (x == keyj).astype(jnp.float32)
                cnt = jnp.dot(eq, ones_col,
                              preferred_element_type=jnp.float32,
                              precision=lax.Precision.HIGHEST)
                is_new = jnp.where(
                    (cnt == float(IN_DIM)) & (found_sc[...] == 0.0),
                    1.0, 0.0)
                found_sc[...] = found_sc[...] + is_new
                fix_sc[...] = fix_sc[...] + is_new * vals_ref[pl.ds(j, 1), :]
                return carry

            lax.fori_loop(0, K_FIXED, scan_key, 0)
            out_ref[...] = jnp.where(found_sc[...] > 0.0, fix_sc[...], net)


@jax.jit
def kernel(x, fixed_keys, fixed_values, W1, b1, W2, b2):
    full = lambda i: (0, 0)
    return pl.pallas_call(
        _body,
        grid=(GRID,),
        in_specs=[
            pl.BlockSpec((BLK, IN_DIM), lambda i: (i, 0)),
            pl.BlockSpec((K_FIXED, IN_DIM), full),
            pl.BlockSpec((IN_DIM, K_FIXED), full),
            pl.BlockSpec((K_FIXED, EMB_DIM), full),
            pl.BlockSpec((IN_DIM, HIDDEN), full),
            pl.BlockSpec((1, HIDDEN), full),
            pl.BlockSpec((HIDDEN, EMB_DIM), full),
            pl.BlockSpec((1, EMB_DIM), full),
        ],
        out_specs=pl.BlockSpec((BLK, EMB_DIM), lambda i: (i, 0)),
        out_shape=jax.ShapeDtypeStruct((B, EMB_DIM), jnp.float32),
        scratch_shapes=[
            pltpu.VMEM((BLK, 1), jnp.float32),
            pltpu.VMEM((BLK, EMB_DIM), jnp.float32),
        ],
        compiler_params=pltpu.CompilerParams(
            dimension_semantics=("arbitrary",)),
    )(x, fixed_keys, fixed_keys.T, fixed_values,
      W1, b1.reshape(1, HIDDEN), W2, b2.reshape(1, EMB_DIM))
